# baseline (speedup 1.0000x reference)
import functools

import jax
import jax.numpy as jnp
from jax import lax
from jax.experimental import pallas as pl
from jax.experimental.pallas import tpu as pltpu
from jax.experimental.pallas import tpu_sc as plsc

N_PRED = 100000
N_AND = 50000
N_OR = 25000
N_PLAN0 = 25000
N_PLAN1 = 50000
E_PA = 200000
E_AO = 100000
E_PLAN = 100000
D_PRED = 128
D_PLAN = 64
H_PRED = 64
H = 128

NC, NS, L = 2, 16, 16
NW = NC * NS
M = 112


def _round_up(x, m):
    return (x + m - 1) // m * m


def _sc_mesh():
    return plsc.VectorSubcoreMesh(
        core_axis_name="c", subcore_axis_name="s", num_cores=NC)


_SC_PARAMS = pltpu.CompilerParams(
    needs_layout_passes=False, use_tc_tiling_on_sc=False)


def _scan_compact(ebuf, store_src, store_loc, base_lo, base_hi, n_vecs):
    lane15 = jnp.full((L,), 15, jnp.int32)

    def scan_body(i, cnt_v):
        s = ebuf[0, pl.ds(i * L, L)]
        d = ebuf[1, pl.ds(i * L, L)]
        m = (d >= base_lo) & (d < base_hi)
        mi = jnp.where(m, 1, 0).astype(jnp.int32)
        pos = plsc.cumsum(mi)
        idx = cnt_v + pos - 1
        store_src(idx, s, m)
        store_loc(idx, d, m)
        return cnt_v + pos.at[lane15].get(mode="promise_in_bounds")

    cnt_v = lax.fori_loop(0, n_vecs, scan_body,
                          jnp.zeros((L,), jnp.int32), unroll=2)
    return jnp.max(cnt_v)


def _make_seg_minmax(E, N_dst, D, kind, C, MB):
    assert E % C == 0 and C % L == 0 and C % 8 == 0
    assert MB & (MB - 1) == 0 and MB <= 128
    SH = MB.bit_length() - 1
    R = _round_up(-(-N_dst // NW), 8)
    Np = R * NW
    AW = _round_up(R * D + D, L * 8)
    n_chunks = E // C
    kmax = -(-C // MB)
    init = {"min": jnp.inf, "max": -jnp.inf}[kind]

    edge_set = [pltpu.VMEM((2, C), jnp.int32),
                pltpu.SemaphoreType.DMA]
    gather_set = [pltpu.VMEM((kmax, MB), jnp.int32),
                  pltpu.VMEM((C + MB,), jnp.int32),
                  pltpu.VMEM((MB, D), jnp.float32),
                  pltpu.VMEM((MB,), jnp.int32),
                  pltpu.SemaphoreType.DMA]

    @functools.partial(
        pl.kernel,
        mesh=_sc_mesh(),
        compiler_params=_SC_PARAMS,
        out_type=jax.ShapeDtypeStruct((Np * D,), jnp.float32),
        scratch_types=(edge_set + edge_set + gather_set + gather_set
                       + [pltpu.VMEM((AW,), jnp.float32)]),
    )
    def seg_kernel(edges_hbm, tab_hbm, out_hbm,
                   ebuf0, seme0, ebuf1, seme1,
                   msrc0, mloc0, msg0, idx0, semg0,
                   msrc1, mloc1, msg1, idx1, semg1, acc):
        E0 = (ebuf0, seme0)
        E1 = (ebuf1, seme1)
        G0 = (msrc0, mloc0, msg0, idx0, semg0)
        G1 = (msrc1, mloc1, msg1, idx1, semg1)

        wid = lax.axis_index("s") * NC + lax.axis_index("c")
        lo = wid * R
        iota = lax.broadcasted_iota(jnp.int32, (L,), 0)
        init_v = jnp.full((L,), init, jnp.float32)

        def init_body(i, _):
            for k in range(8):
                acc[pl.ds(i * (L * 8) + k * L, L)] = init_v
            return 0

        lax.fori_loop(0, AW // (L * 8), init_body, 0)

        def edge_fetch(c, eset):
            ebuf, sem = eset
            coff = pl.multiple_of(c * C, 8)
            pltpu.async_copy(edges_hbm.at[:, pl.ds(coff, C)], ebuf, sem)

        def edge_wait(eset):
            ebuf, sem = eset
            pltpu.make_async_copy(
                edges_hbm.at[:, pl.ds(0, C)], ebuf, sem).wait()

        def issue(j, gset):
            msrc, _, msg, idx, sem = gset
            for k in range(MB // L):
                idx[pl.ds(k * L, L)] = msrc[j, pl.ds(k * L, L)]
            pltpu.async_copy(tab_hbm.at[idx], msg, sem)

        def wait_g(gset):
            _, _, msg, idx, sem = gset
            pltpu.make_async_copy(tab_hbm.at[idx], msg, sem).wait()

        def rmw(j, gset):
            _, mloc, msg, _, _ = gset

            def edge_body(e, _):
                bvec = plsc.load_gather(
                    mloc, [jnp.broadcast_to(j * MB + e, (L,))])
                base = bvec[0]
                for f in range(D // L):
                    mv = msg[e, pl.ds(f * L, L)]
                    o = pl.ds(base + f * L, L)
                    av = acc[o]
                    acc[o] = (jnp.minimum(av, mv) if kind == "min"
                              else jnp.maximum(av, mv))
                return 0

            lax.fori_loop(0, MB, edge_body, 0, unroll=2)

        def scan_chunk(eset, gset):
            ebuf, _ = eset
            msrc, mloc, _, _, _ = gset

            def store_loc(idx, d, m):
                plsc.store_scatter(mloc, [idx], (d - lo) * D, mask=m)

            def store_src(idx, s, m):
                plsc.store_scatter(msrc, [idx >> SH, idx & (MB - 1)], s,
                                   mask=m)

            cnt = _scan_compact(ebuf, store_src, store_loc,
                                lo, lo + R, C // L)

            nb = (cnt + (MB - 1)) // MB
            total = nb * MB
            for k in range(MB // L):
                pidx = cnt + k * L + iota
                pm = pidx < total
                plsc.store_scatter(msrc, [pidx >> SH, pidx & (MB - 1)],
                                   jnp.zeros((L,), jnp.int32), mask=pm)
                plsc.store_scatter(mloc, [pidx],
                                   jnp.full((L,), R * D, jnp.int32), mask=pm)
            return nb

        def drain_prev(nb_prev, gset):
            def batch_body(j, _):
                wait_g(gset)
                rmw(j, gset)

                @pl.when(j + 1 < nb_prev)
                def _():
                    issue(j + 1, gset)

                return 0

            lax.fori_loop(0, nb_prev, batch_body, 0)

        def do_chunk(c, nb_old, e_cur, e_nxt, g_cur, g_old):
            edge_wait(e_cur)
            nb = scan_chunk(e_cur, g_cur)

            @pl.when(nb > 0)
            def _():
                issue(0, g_cur)

            @pl.when(c + 1 < n_chunks)
            def _():
                edge_fetch(c + 1, e_nxt)

            drain_prev(nb_old, g_old)
            return nb

        edge_fetch(0, E0)

        def chunk_body(c, nb_prev):
            return lax.cond(
                c % 2 == 0,
                lambda: do_chunk(c, nb_prev, E0, E1, G0, G1),
                lambda: do_chunk(c, nb_prev, E1, E0, G1, G0),
            )

        nb_last = lax.fori_loop(0, n_chunks, chunk_body, jnp.int32(0))
        if (n_chunks - 1) % 2 == 0:
            drain_prev(nb_last, G0)
        else:
            drain_prev(nb_last, G1)

        bad = jnp.float32(init)

        def fin_body(i, _):
            for k in range(4):
                o = pl.ds(i * (L * 4) + k * L, L)
                v = acc[o]
                acc[o] = jnp.where(v == bad, 0.0, v)
            return 0

        lax.fori_loop(0, (R * D) // (L * 4), fin_body, 0)

        pltpu.sync_copy(acc.at[pl.ds(0, R * D)],
                        out_hbm.at[pl.ds(lo * D, R * D)])

    return seg_kernel, Np


def _make_seg_sum(E, N_dst, D, C):
    MS = 128
    assert E % C == 0 and C % L == 0 and C % 8 == 0
    half = _round_up(-(-N_dst // NC), NS * 8)
    Np = half * NC
    per_tile = _round_up(-(-(half + 1) // NS), 8)
    alloc = per_tile * NS
    wb = half // NS
    assert wb * NS == half
    n_chunks = E // C
    kmax = C // MS + 1

    def _tiled(n):
        return [(i * MS, min(MS, n - i * MS)) for i in range(-(-n // MS))]

    @functools.partial(
        pl.kernel,
        mesh=_sc_mesh(),
        compiler_params=_SC_PARAMS,
        out_type=jax.ShapeDtypeStruct((Np, D), jnp.float32),
        scratch_types=[
            pltpu.VMEM((2, C), jnp.int32),
            pltpu.VMEM((kmax, MS), jnp.int32),
            pltpu.VMEM((kmax, MS), jnp.int32),
            pltpu.VMEM((MS, D), jnp.float32),
            pltpu.VMEM((MS,), jnp.int32),
            pltpu.VMEM((MS,), jnp.int32),
            pltpu.VMEM_SHARED((alloc, D), jnp.float32),
            pltpu.SemaphoreType.DMA,
            pltpu.SemaphoreType.DMA,
        ],
    )
    def sum_kernel(zero_hbm, edges_hbm, tab_hbm, out_hbm,
                   ebuf, msrc, mdst, msg, idx_g, idx_s,
                   acc, sem_a, sem_b):
        cid = lax.axis_index("c")
        sid = lax.axis_index("s")
        base = cid * half
        iota = lax.broadcasted_iota(jnp.int32, (L,), 0)

        pltpu.sync_copy(zero_hbm, msg)
        for off, rows in _tiled(per_tile):
            pltpu.sync_copy(
                msg.at[pl.ds(0, rows)],
                acc.at[pl.ds(sid * per_tile + off, rows)])
        plsc.subcore_barrier()

        nmy = (n_chunks - sid + NS - 1) // NS

        def chunk_body(t, _):
            c = sid + t * NS
            coff = pl.multiple_of(c * C, 8)
            pltpu.async_copy(
                edges_hbm.at[:, pl.ds(coff, C)], ebuf, sem_a).wait()

            def store_loc(idx, d, m):
                plsc.store_scatter(
                    mdst, [idx >> 7, idx & (MS - 1)], d - base, mask=m)

            def store_src(idx, s, m):
                plsc.store_scatter(
                    msrc, [idx >> 7, idx & (MS - 1)], s, mask=m)

            cnt = _scan_compact(ebuf, store_src, store_loc,
                                base, base + half, C // L)

            nb = (cnt + (MS - 1)) // MS
            total = nb * MS
            for k in range(MS // L):
                pidx = cnt + k * L + iota
                pm = pidx < total
                plsc.store_scatter(msrc, [pidx >> 7, pidx & (MS - 1)],
                                   jnp.zeros((L,), jnp.int32), mask=pm)
                plsc.store_scatter(mdst, [pidx >> 7, pidx & (MS - 1)],
                                   jnp.full((L,), half, jnp.int32), mask=pm)

            def batch_body(j, _):
                for k in range(MS // L):
                    idx_g[pl.ds(k * L, L)] = msrc[j, pl.ds(k * L, L)]
                    idx_s[pl.ds(k * L, L)] = mdst[j, pl.ds(k * L, L)]
                pltpu.async_copy(tab_hbm.at[idx_g], msg, sem_b).wait()
                pltpu.sync_copy(msg, acc.at[idx_s], add=True)
                return 0

            lax.fori_loop(0, nb, batch_body, 0)
            return 0

        lax.fori_loop(0, nmy, chunk_body, 0)
        plsc.subcore_barrier()

        for off, rows in _tiled(wb):
            pltpu.sync_copy(
                acc.at[pl.ds(sid * wb + off, rows)],
                msg.at[pl.ds(0, rows)])
            pltpu.sync_copy(
                msg.at[pl.ds(0, rows)],
                out_hbm.at[pl.ds(base + sid * wb + off, rows)])

    return sum_kernel, Np


def _make_gather(B, D):
    bpw = B // NW
    assert bpw % M == 0

    @functools.partial(
        pl.kernel,
        mesh=_sc_mesh(),
        compiler_params=_SC_PARAMS,
        out_type=jax.ShapeDtypeStruct((B, D), jnp.float32),
        scratch_types=[
            pltpu.VMEM((M,), jnp.int32),
            pltpu.VMEM((M, D), jnp.float32),
            pltpu.SemaphoreType.DMA,
        ],
    )
    def gather_kernel(tab_hbm, idx_hbm, out_hbm, idx_v, rows_v, sem):
        wid = lax.axis_index("s") * NC + lax.axis_index("c")
        base = wid * bpw

        def body(j, _):
            off = pl.multiple_of(base + j * M, 8)
            pltpu.sync_copy(idx_hbm.at[pl.ds(off, M)], idx_v)
            pltpu.async_copy(tab_hbm.at[idx_v], rows_v, sem).wait()
            pltpu.sync_copy(rows_v, out_hbm.at[pl.ds(off, M)])
            return 0

        lax.fori_loop(0, bpw // M, body, 0)

    return gather_kernel


def _stats_body(x_ref, s_ref, q_ref):
    @pl.when(pl.program_id(0) == 0)
    def _():
        s_ref[...] = jnp.zeros_like(s_ref)
        q_ref[...] = jnp.zeros_like(q_ref)

    x = x_ref[...]
    s_ref[...] += jnp.sum(x, axis=0, keepdims=True)
    q_ref[...] += jnp.sum(x * x, axis=0, keepdims=True)


def _pred_enc_body(x_ref, s_ref, q_ref, g_ref, bt_ref, w_ref, b_ref, o_ref):
    n = jnp.float32(N_PRED)
    mu = s_ref[...] / n
    var = q_ref[...] / n - mu * mu
    scale = g_ref[...][None, :] * jax.lax.rsqrt(var + 1e-5)
    shift = bt_ref[...][None, :] - mu * scale
    xn = x_ref[...] * scale + shift
    o_ref[...] = jax.nn.relu(xn @ w_ref[...] + b_ref[...][None, :])


def _enc1_body(pf_ref, pph_ref, w_ref, b_ref, o_ref):
    e = jax.nn.relu(pf_ref[...] @ w_ref[...] + b_ref[...][None, :])
    o_ref[...] = jnp.concatenate([e, pph_ref[...]], axis=1)


def _final_body(pf_ref, pph_ref, agg_ref, wp_ref, bp_ref,
                w1_ref, b1_ref, w2_ref, b2_ref, w3_ref, b3_ref, o_ref):
    e = jax.nn.relu(pf_ref[...] @ wp_ref[...] + bp_ref[...][None, :])
    h0 = jnp.concatenate([e, pph_ref[...]], axis=1) + agg_ref[...]
    h = jax.nn.relu(h0 @ w1_ref[...] + b1_ref[...][None, :])
    h = jax.nn.relu(h @ w2_ref[...] + b2_ref[...][None, :])
    o_ref[...] = h @ w3_ref[...] + b3_ref[...][None, :]


def kernel(pred_feat, plan_feat0, plan_feat1, src_pred, dst_and, src_and,
           dst_or, map0, map1, src_plan1, dst_plan0, bn_gamma, bn_beta,
           W_pred, b_pred, W_plan, b_plan, W1, b1, W2, b2, W3, b3):
    f32 = jnp.float32

    RB = 1000
    sums, sumsq = pl.pallas_call(
        _stats_body,
        grid=(N_PRED // RB,),
        in_specs=[pl.BlockSpec((RB, D_PRED), lambda i: (i, 0))],
        out_specs=[pl.BlockSpec((1, D_PRED), lambda i: (0, 0)),
                   pl.BlockSpec((1, D_PRED), lambda i: (0, 0))],
        out_shape=[jax.ShapeDtypeStruct((1, D_PRED), f32),
                   jax.ShapeDtypeStruct((1, D_PRED), f32)],
    )(pred_feat)

    pred_enc = pl.pallas_call(
        _pred_enc_body,
        grid=(N_PRED // RB,),
        in_specs=[
            pl.BlockSpec((RB, D_PRED), lambda i: (i, 0)),
            pl.BlockSpec((1, D_PRED), lambda i: (0, 0)),
            pl.BlockSpec((1, D_PRED), lambda i: (0, 0)),
            pl.BlockSpec((D_PRED,), lambda i: (0,)),
            pl.BlockSpec((D_PRED,), lambda i: (0,)),
            pl.BlockSpec((D_PRED, H_PRED), lambda i: (0, 0)),
            pl.BlockSpec((H_PRED,), lambda i: (0,)),
        ],
        out_specs=pl.BlockSpec((RB, H_PRED), lambda i: (i, 0)),
        out_shape=jax.ShapeDtypeStruct((N_PRED, H_PRED), f32),
    )(pred_feat, sums, sumsq, bn_gamma, bn_beta, W_pred, b_pred)

    seg_min, np_and = _make_seg_minmax(E_PA, N_AND, H_PRED, "min", 2000, 64)
    and_h = seg_min(jnp.stack([src_pred, dst_and]),
                    pred_enc).reshape(np_and, H_PRED)

    seg_max, np_or = _make_seg_minmax(E_AO, N_OR, H_PRED, "max", 4000, 64)
    or_h = seg_max(jnp.stack([src_and, dst_or]),
                   and_h).reshape(np_or, H_PRED)

    B0 = _round_up(N_PLAN0, NW * M)
    B1 = _round_up(N_PLAN1, NW * M)
    map0p = jnp.pad(map0, (0, B0 - N_PLAN0))
    map1p = jnp.pad(map1, (0, B1 - N_PLAN1))
    pph0 = _make_gather(B0, H_PRED)(or_h, map0p)
    pph1 = _make_gather(B1, H_PRED)(pred_enc, map1p)

    RB1 = 1000
    enc1 = pl.pallas_call(
        _enc1_body,
        grid=(N_PLAN1 // RB1,),
        in_specs=[
            pl.BlockSpec((RB1, D_PLAN), lambda i: (i, 0)),
            pl.BlockSpec((RB1, H_PRED), lambda i: (i, 0)),
            pl.BlockSpec((D_PLAN, H_PRED), lambda i: (0, 0)),
            pl.BlockSpec((H_PRED,), lambda i: (0,)),
        ],
        out_specs=pl.BlockSpec((RB1, H), lambda i: (i, 0)),
        out_shape=jax.ShapeDtypeStruct((N_PLAN1, H), f32),
    )(plan_feat1, pph1, W_plan, b_plan)

    seg_sum, np_p0 = _make_seg_sum(E_PLAN, N_PLAN0, H, 2000)
    agg = seg_sum(jnp.zeros((128, H), f32),
                  jnp.stack([src_plan1, dst_plan0]), enc1)

    RB0 = 1000
    out = pl.pallas_call(
        _final_body,
        grid=(N_PLAN0 // RB0,),
        in_specs=[
            pl.BlockSpec((RB0, D_PLAN), lambda i: (i, 0)),
            pl.BlockSpec((RB0, H_PRED), lambda i: (i, 0)),
            pl.BlockSpec((RB0, H), lambda i: (i, 0)),
            pl.BlockSpec((D_PLAN, H_PRED), lambda i: (0, 0)),
            pl.BlockSpec((H_PRED,), lambda i: (0,)),
            pl.BlockSpec((H, H), lambda i: (0, 0)),
            pl.BlockSpec((H,), lambda i: (0,)),
            pl.BlockSpec((H, H), lambda i: (0, 0)),
            pl.BlockSpec((H,), lambda i: (0,)),
            pl.BlockSpec((H, 1), lambda i: (0, 0)),
            pl.BlockSpec((1,), lambda i: (0,)),
        ],
        out_specs=pl.BlockSpec((RB0, 1), lambda i: (i, 0)),
        out_shape=jax.ShapeDtypeStruct((N_PLAN0, 1), f32),
    )(plan_feat0, pph0, agg, W_plan, b_plan, W1, b1, W2, b2, W3, b3)
    return out


# speedup vs baseline: 1.1801x; 1.1801x over previous
import functools

import jax
import jax.numpy as jnp
from jax import lax
from jax.experimental import pallas as pl
from jax.experimental.pallas import tpu as pltpu
from jax.experimental.pallas import tpu_sc as plsc

N_PRED = 100000
N_AND = 50000
N_OR = 25000
N_PLAN0 = 25000
N_PLAN1 = 50000
E_PA = 200000
E_AO = 100000
E_PLAN = 100000
D_PRED = 128
D_PLAN = 64
H_PRED = 64
H = 128

NC, NS, L = 2, 16, 16
NW = NC * NS
M = 112


def _round_up(x, m):
    return (x + m - 1) // m * m


def _sc_mesh():
    return plsc.VectorSubcoreMesh(
        core_axis_name="c", subcore_axis_name="s", num_cores=NC)


_SC_PARAMS = pltpu.CompilerParams(
    needs_layout_passes=False, use_tc_tiling_on_sc=False)


def _scan_compact(ebuf, store_src, store_loc, base_lo, base_hi, n_vecs):
    lane15 = jnp.full((L,), 15, jnp.int32)

    def scan_body(i, cnt_v):
        s = ebuf[0, pl.ds(i * L, L)]
        d = ebuf[1, pl.ds(i * L, L)]
        m = (d >= base_lo) & (d < base_hi)
        mi = jnp.where(m, 1, 0).astype(jnp.int32)
        pos = plsc.cumsum(mi)
        idx = cnt_v + pos - 1
        store_src(idx, s, m)
        store_loc(idx, d, m)
        return cnt_v + pos.at[lane15].get(mode="promise_in_bounds")

    cnt_v = lax.fori_loop(0, n_vecs, scan_body,
                          jnp.zeros((L,), jnp.int32), unroll=2)
    return jnp.max(cnt_v)


def _make_seg_minmax(E, N_dst, D, kind, C, MB):
    assert E % C == 0 and C % L == 0 and C % 8 == 0
    assert MB & (MB - 1) == 0 and MB <= 128
    SH = MB.bit_length() - 1
    R = _round_up(-(-N_dst // NW), 8)
    Np = R * NW
    AW = _round_up(R * D + D, L * 8)
    n_chunks = E // C
    kmax = -(-C // MB)
    init = {"min": jnp.inf, "max": -jnp.inf}[kind]

    edge_set = [pltpu.VMEM((2, C), jnp.int32),
                pltpu.SemaphoreType.DMA]
    gather_set = [pltpu.VMEM((kmax, MB), jnp.int32),
                  pltpu.VMEM((C + MB,), jnp.int32),
                  pltpu.VMEM((MB, D), jnp.float32),
                  pltpu.VMEM((MB,), jnp.int32),
                  pltpu.SemaphoreType.DMA]

    @functools.partial(
        pl.kernel,
        mesh=_sc_mesh(),
        compiler_params=_SC_PARAMS,
        out_type=jax.ShapeDtypeStruct((Np * D,), jnp.float32),
        scratch_types=(edge_set + edge_set + gather_set + gather_set
                       + gather_set + [pltpu.VMEM((AW,), jnp.float32)]),
    )
    def seg_kernel(edges_hbm, tab_hbm, out_hbm,
                   ebuf0, seme0, ebuf1, seme1,
                   msrc0, mloc0, msg0, idx0, semg0,
                   msrc1, mloc1, msg1, idx1, semg1,
                   msrc2, mloc2, msg2, idx2, semg2, acc):
        E0 = (ebuf0, seme0)
        E1 = (ebuf1, seme1)
        G0 = (msrc0, mloc0, msg0, idx0, semg0)
        G1 = (msrc1, mloc1, msg1, idx1, semg1)
        G2 = (msrc2, mloc2, msg2, idx2, semg2)

        wid = lax.axis_index("s") * NC + lax.axis_index("c")
        lo = wid * R
        iota = lax.broadcasted_iota(jnp.int32, (L,), 0)
        init_v = jnp.full((L,), init, jnp.float32)

        def init_body(i, _):
            for k in range(8):
                acc[pl.ds(i * (L * 8) + k * L, L)] = init_v
            return 0

        lax.fori_loop(0, AW // (L * 8), init_body, 0)

        def edge_fetch(c, eset):
            ebuf, sem = eset
            coff = pl.multiple_of(c * C, 8)
            pltpu.async_copy(edges_hbm.at[:, pl.ds(coff, C)], ebuf, sem)

        def edge_wait(eset):
            ebuf, sem = eset
            pltpu.make_async_copy(
                edges_hbm.at[:, pl.ds(0, C)], ebuf, sem).wait()

        def issue(j, gset):
            msrc, _, msg, idx, sem = gset
            for k in range(MB // L):
                idx[pl.ds(k * L, L)] = msrc[j, pl.ds(k * L, L)]
            pltpu.async_copy(tab_hbm.at[idx], msg, sem)

        def wait_g(gset):
            _, _, msg, idx, sem = gset
            pltpu.make_async_copy(tab_hbm.at[idx], msg, sem).wait()

        def rmw(j, gset):
            _, mloc, msg, _, _ = gset

            def edge_body(e, _):
                bvec = plsc.load_gather(
                    mloc, [jnp.broadcast_to(j * MB + e, (L,))])
                base = bvec[0]
                for f in range(D // L):
                    mv = msg[e, pl.ds(f * L, L)]
                    o = pl.ds(base + f * L, L)
                    av = acc[o]
                    acc[o] = (jnp.minimum(av, mv) if kind == "min"
                              else jnp.maximum(av, mv))
                return 0

            lax.fori_loop(0, MB, edge_body, 0, unroll=2)

        def scan_chunk(eset, gset):
            ebuf, _ = eset
            msrc, mloc, _, _, _ = gset

            def store_loc(idx, d, m):
                plsc.store_scatter(mloc, [idx], (d - lo) * D, mask=m)

            def store_src(idx, s, m):
                plsc.store_scatter(msrc, [idx >> SH, idx & (MB - 1)], s,
                                   mask=m)

            cnt = _scan_compact(ebuf, store_src, store_loc,
                                lo, lo + R, C // L)

            nb = (cnt + (MB - 1)) // MB
            total = nb * MB
            for k in range(MB // L):
                pidx = cnt + k * L + iota
                pm = pidx < total
                plsc.store_scatter(msrc, [pidx >> SH, pidx & (MB - 1)],
                                   jnp.zeros((L,), jnp.int32), mask=pm)
                plsc.store_scatter(mloc, [pidx],
                                   jnp.full((L,), R * D, jnp.int32), mask=pm)
            return nb

        def drain_prev(nb_prev, gset):
            def batch_body(j, _):
                wait_g(gset)
                rmw(j, gset)

                @pl.when(j + 1 < nb_prev)
                def _():
                    issue(j + 1, gset)

                return 0

            lax.fori_loop(0, nb_prev, batch_body, 0)

        def do_chunk(c, nb_old, e_cur, e_nxt, g_cur, g_old):
            edge_wait(e_cur)
            nb = scan_chunk(e_cur, g_cur)

            @pl.when(nb > 0)
            def _():
                issue(0, g_cur)

            @pl.when(c + 1 < n_chunks)
            def _():
                edge_fetch(c + 1, e_nxt)

            drain_prev(nb_old, g_old)
            return nb

        edge_fetch(0, E0)
        G = (G0, G1, G2)

        def chunk_body(c, carry):
            nb1, nb2 = carry
            nb = lax.switch(
                c % 6,
                [
                    lambda: do_chunk(c, nb2, E0, E1, G0, G1),
                    lambda: do_chunk(c, nb2, E1, E0, G1, G2),
                    lambda: do_chunk(c, nb2, E0, E1, G2, G0),
                    lambda: do_chunk(c, nb2, E1, E0, G0, G1),
                    lambda: do_chunk(c, nb2, E0, E1, G1, G2),
                    lambda: do_chunk(c, nb2, E1, E0, G2, G0),
                ],
            )
            return nb, nb1

        nb1, nb2 = lax.fori_loop(
            0, n_chunks, chunk_body, (jnp.int32(0), jnp.int32(0)))
        drain_prev(nb2, G[(n_chunks - 2) % 3])
        drain_prev(nb1, G[(n_chunks - 1) % 3])

        bad = jnp.float32(init)

        def fin_body(i, _):
            for k in range(4):
                o = pl.ds(i * (L * 4) + k * L, L)
                v = acc[o]
                acc[o] = jnp.where(v == bad, 0.0, v)
            return 0

        lax.fori_loop(0, (R * D) // (L * 4), fin_body, 0)

        pltpu.sync_copy(acc.at[pl.ds(0, R * D)],
                        out_hbm.at[pl.ds(lo * D, R * D)])

    return seg_kernel, Np


def _make_seg_sum(E, N_dst, D, C):
    MS = 128
    assert E % C == 0 and C % L == 0 and C % 8 == 0
    half = _round_up(-(-N_dst // NC), NS * 8)
    Np = half * NC
    per_tile = _round_up(-(-(half + 1) // NS), 8)
    alloc = per_tile * NS
    wb = half // NS
    assert wb * NS == half
    n_chunks = E // C
    kmax = C // MS + 1

    def _tiled(n):
        return [(i * MS, min(MS, n - i * MS)) for i in range(-(-n // MS))]

    @functools.partial(
        pl.kernel,
        mesh=_sc_mesh(),
        compiler_params=_SC_PARAMS,
        out_type=jax.ShapeDtypeStruct((Np, D), jnp.float32),
        scratch_types=[
            pltpu.VMEM((2, C), jnp.int32),
            pltpu.VMEM((kmax, MS), jnp.int32),
            pltpu.VMEM((kmax, MS), jnp.int32),
            pltpu.VMEM((MS, D), jnp.float32),
            pltpu.VMEM((MS,), jnp.int32),
            pltpu.VMEM((MS,), jnp.int32),
            pltpu.VMEM_SHARED((alloc, D), jnp.float32),
            pltpu.SemaphoreType.DMA,
            pltpu.SemaphoreType.DMA,
        ],
    )
    def sum_kernel(zero_hbm, edges_hbm, tab_hbm, out_hbm,
                   ebuf, msrc, mdst, msg, idx_g, idx_s,
                   acc, sem_a, sem_b):
        cid = lax.axis_index("c")
        sid = lax.axis_index("s")
        base = cid * half
        iota = lax.broadcasted_iota(jnp.int32, (L,), 0)

        pltpu.sync_copy(zero_hbm, msg)
        for off, rows in _tiled(per_tile):
            pltpu.sync_copy(
                msg.at[pl.ds(0, rows)],
                acc.at[pl.ds(sid * per_tile + off, rows)])
        plsc.subcore_barrier()

        nmy = (n_chunks - sid + NS - 1) // NS

        def chunk_body(t, _):
            c = sid + t * NS
            coff = pl.multiple_of(c * C, 8)
            pltpu.async_copy(
                edges_hbm.at[:, pl.ds(coff, C)], ebuf, sem_a).wait()

            def store_loc(idx, d, m):
                plsc.store_scatter(
                    mdst, [idx >> 7, idx & (MS - 1)], d - base, mask=m)

            def store_src(idx, s, m):
                plsc.store_scatter(
                    msrc, [idx >> 7, idx & (MS - 1)], s, mask=m)

            cnt = _scan_compact(ebuf, store_src, store_loc,
                                base, base + half, C // L)

            nb = (cnt + (MS - 1)) // MS
            total = nb * MS
            for k in range(MS // L):
                pidx = cnt + k * L + iota
                pm = pidx < total
                plsc.store_scatter(msrc, [pidx >> 7, pidx & (MS - 1)],
                                   jnp.zeros((L,), jnp.int32), mask=pm)
                plsc.store_scatter(mdst, [pidx >> 7, pidx & (MS - 1)],
                                   jnp.full((L,), half, jnp.int32), mask=pm)

            def batch_body(j, _):
                for k in range(MS // L):
                    idx_g[pl.ds(k * L, L)] = msrc[j, pl.ds(k * L, L)]
                    idx_s[pl.ds(k * L, L)] = mdst[j, pl.ds(k * L, L)]
                pltpu.async_copy(tab_hbm.at[idx_g], msg, sem_b).wait()
                pltpu.sync_copy(msg, acc.at[idx_s], add=True)
                return 0

            lax.fori_loop(0, nb, batch_body, 0)
            return 0

        lax.fori_loop(0, nmy, chunk_body, 0)
        plsc.subcore_barrier()

        for off, rows in _tiled(wb):
            pltpu.sync_copy(
                acc.at[pl.ds(sid * wb + off, rows)],
                msg.at[pl.ds(0, rows)])
            pltpu.sync_copy(
                msg.at[pl.ds(0, rows)],
                out_hbm.at[pl.ds(base + sid * wb + off, rows)])

    return sum_kernel, Np


def _make_gather(B, D):
    bpw = B // NW
    assert bpw % M == 0

    @functools.partial(
        pl.kernel,
        mesh=_sc_mesh(),
        compiler_params=_SC_PARAMS,
        out_type=jax.ShapeDtypeStruct((B, D), jnp.float32),
        scratch_types=[
            pltpu.VMEM((M,), jnp.int32),
            pltpu.VMEM((M, D), jnp.float32),
            pltpu.SemaphoreType.DMA,
        ],
    )
    def gather_kernel(tab_hbm, idx_hbm, out_hbm, idx_v, rows_v, sem):
        wid = lax.axis_index("s") * NC + lax.axis_index("c")
        base = wid * bpw

        def body(j, _):
            off = pl.multiple_of(base + j * M, 8)
            pltpu.sync_copy(idx_hbm.at[pl.ds(off, M)], idx_v)
            pltpu.async_copy(tab_hbm.at[idx_v], rows_v, sem).wait()
            pltpu.sync_copy(rows_v, out_hbm.at[pl.ds(off, M)])
            return 0

        lax.fori_loop(0, bpw // M, body, 0)

    return gather_kernel


def _stats_body(x_ref, s_ref, q_ref):
    @pl.when(pl.program_id(0) == 0)
    def _():
        s_ref[...] = jnp.zeros_like(s_ref)
        q_ref[...] = jnp.zeros_like(q_ref)

    x = x_ref[...]
    s_ref[...] += jnp.sum(x, axis=0, keepdims=True)
    q_ref[...] += jnp.sum(x * x, axis=0, keepdims=True)


def _pred_enc_body(x_ref, s_ref, q_ref, g_ref, bt_ref, w_ref, b_ref, o_ref):
    n = jnp.float32(N_PRED)
    mu = s_ref[...] / n
    var = q_ref[...] / n - mu * mu
    scale = g_ref[...][None, :] * jax.lax.rsqrt(var + 1e-5)
    shift = bt_ref[...][None, :] - mu * scale
    xn = x_ref[...] * scale + shift
    o_ref[...] = jax.nn.relu(xn @ w_ref[...] + b_ref[...][None, :])


def _enc1_body(pf_ref, pph_ref, w_ref, b_ref, o_ref):
    e = jax.nn.relu(pf_ref[...] @ w_ref[...] + b_ref[...][None, :])
    o_ref[...] = jnp.concatenate([e, pph_ref[...]], axis=1)


def _final_body(pf_ref, pph_ref, agg_ref, wp_ref, bp_ref,
                w1_ref, b1_ref, w2_ref, b2_ref, w3_ref, b3_ref, o_ref):
    e = jax.nn.relu(pf_ref[...] @ wp_ref[...] + bp_ref[...][None, :])
    h0 = jnp.concatenate([e, pph_ref[...]], axis=1) + agg_ref[...]
    h = jax.nn.relu(h0 @ w1_ref[...] + b1_ref[...][None, :])
    h = jax.nn.relu(h @ w2_ref[...] + b2_ref[...][None, :])
    o_ref[...] = h @ w3_ref[...] + b3_ref[...][None, :]


def kernel(pred_feat, plan_feat0, plan_feat1, src_pred, dst_and, src_and,
           dst_or, map0, map1, src_plan1, dst_plan0, bn_gamma, bn_beta,
           W_pred, b_pred, W_plan, b_plan, W1, b1, W2, b2, W3, b3):
    f32 = jnp.float32

    RB = 1000
    sums, sumsq = pl.pallas_call(
        _stats_body,
        grid=(N_PRED // RB,),
        in_specs=[pl.BlockSpec((RB, D_PRED), lambda i: (i, 0))],
        out_specs=[pl.BlockSpec((1, D_PRED), lambda i: (0, 0)),
                   pl.BlockSpec((1, D_PRED), lambda i: (0, 0))],
        out_shape=[jax.ShapeDtypeStruct((1, D_PRED), f32),
                   jax.ShapeDtypeStruct((1, D_PRED), f32)],
    )(pred_feat)

    pred_enc = pl.pallas_call(
        _pred_enc_body,
        grid=(N_PRED // RB,),
        in_specs=[
            pl.BlockSpec((RB, D_PRED), lambda i: (i, 0)),
            pl.BlockSpec((1, D_PRED), lambda i: (0, 0)),
            pl.BlockSpec((1, D_PRED), lambda i: (0, 0)),
            pl.BlockSpec((D_PRED,), lambda i: (0,)),
            pl.BlockSpec((D_PRED,), lambda i: (0,)),
            pl.BlockSpec((D_PRED, H_PRED), lambda i: (0, 0)),
            pl.BlockSpec((H_PRED,), lambda i: (0,)),
        ],
        out_specs=pl.BlockSpec((RB, H_PRED), lambda i: (i, 0)),
        out_shape=jax.ShapeDtypeStruct((N_PRED, H_PRED), f32),
    )(pred_feat, sums, sumsq, bn_gamma, bn_beta, W_pred, b_pred)

    seg_min, np_and = _make_seg_minmax(E_PA, N_AND, H_PRED, "min", 1600, 64)
    and_h = seg_min(jnp.stack([src_pred, dst_and]),
                    pred_enc).reshape(np_and, H_PRED)

    seg_max, np_or = _make_seg_minmax(E_AO, N_OR, H_PRED, "max", 4000, 64)
    or_h = seg_max(jnp.stack([src_and, dst_or]),
                   and_h).reshape(np_or, H_PRED)

    B0 = _round_up(N_PLAN0, NW * M)
    B1 = _round_up(N_PLAN1, NW * M)
    map0p = jnp.pad(map0, (0, B0 - N_PLAN0))
    map1p = jnp.pad(map1, (0, B1 - N_PLAN1))
    pph0 = _make_gather(B0, H_PRED)(or_h, map0p)
    pph1 = _make_gather(B1, H_PRED)(pred_enc, map1p)

    RB1 = 1000
    enc1 = pl.pallas_call(
        _enc1_body,
        grid=(N_PLAN1 // RB1,),
        in_specs=[
            pl.BlockSpec((RB1, D_PLAN), lambda i: (i, 0)),
            pl.BlockSpec((RB1, H_PRED), lambda i: (i, 0)),
            pl.BlockSpec((D_PLAN, H_PRED), lambda i: (0, 0)),
            pl.BlockSpec((H_PRED,), lambda i: (0,)),
        ],
        out_specs=pl.BlockSpec((RB1, H), lambda i: (i, 0)),
        out_shape=jax.ShapeDtypeStruct((N_PLAN1, H), f32),
    )(plan_feat1, pph1, W_plan, b_plan)

    seg_sum, np_p0 = _make_seg_sum(E_PLAN, N_PLAN0, H, 2000)
    agg = seg_sum(jnp.zeros((128, H), f32),
                  jnp.stack([src_plan1, dst_plan0]), enc1)

    RB0 = 1000
    out = pl.pallas_call(
        _final_body,
        grid=(N_PLAN0 // RB0,),
        in_specs=[
            pl.BlockSpec((RB0, D_PLAN), lambda i: (i, 0)),
            pl.BlockSpec((RB0, H_PRED), lambda i: (i, 0)),
            pl.BlockSpec((RB0, H), lambda i: (i, 0)),
            pl.BlockSpec((D_PLAN, H_PRED), lambda i: (0, 0)),
            pl.BlockSpec((H_PRED,), lambda i: (0,)),
            pl.BlockSpec((H, H), lambda i: (0, 0)),
            pl.BlockSpec((H,), lambda i: (0,)),
            pl.BlockSpec((H, H), lambda i: (0, 0)),
            pl.BlockSpec((H,), lambda i: (0,)),
            pl.BlockSpec((H, 1), lambda i: (0, 0)),
            pl.BlockSpec((1,), lambda i: (0,)),
        ],
        out_specs=pl.BlockSpec((RB0, 1), lambda i: (i, 0)),
        out_shape=jax.ShapeDtypeStruct((N_PLAN0, 1), f32),
    )(plan_feat0, pph0, agg, W_plan, b_plan, W1, b1, W2, b2, W3, b3)
    return out
